# baseline (speedup 1.0000x reference)
import jax
import jax.numpy as jnp
from jax import lax
from jax.experimental import pallas as pl

_UNK = 22
_THRESH = 838861
_K0 = 0
_K1 = 42
_KS2 = _K0 ^ _K1 ^ 0x1BD11BDA
_ROT = ((13, 15, 26, 6), (17, 29, 16, 24))


def _rotl(v, d):
    return lax.shift_right_logical(v, jnp.int32(32 - d)) | (v << jnp.int32(d))


def _threefry_mask_body(x_ref, o_ref):
    x = x_ref[...]
    r = lax.broadcasted_iota(jnp.int32, x.shape, 0)
    c = lax.broadcasted_iota(jnp.int32, x.shape, 1)
    idx = r * jnp.int32(x.shape[1]) + c
    x0 = jnp.full(x.shape, jnp.int32(_K0), dtype=jnp.int32)
    x1 = idx + jnp.int32(_K1)
    ks = (_K0, _K1, _KS2)
    for i in range(5):
        for d in _ROT[i % 2]:
            x0 = x0 + x1
            x1 = _rotl(x1, d)
            x1 = x1 ^ x0
        x0 = x0 + jnp.int32(ks[(i + 1) % 3])
        x1 = x1 + jnp.int32((ks[(i + 2) % 3] + i + 1) & 0x7FFFFFFF) + jnp.int32(
            -0x80000000 if (ks[(i + 2) % 3] + i + 1) & 0x80000000 else 0
        )
    bits = x0 ^ x1
    mask = lax.shift_right_logical(bits, jnp.int32(9)) < jnp.int32(_THRESH)
    o_ref[...] = jnp.where(mask, jnp.int32(_UNK), x)


@jax.jit
def kernel(x):
    n = x.size
    xr = x.reshape(n // 128, 128)
    out = pl.pallas_call(
        _threefry_mask_body,
        out_shape=jax.ShapeDtypeStruct(xr.shape, xr.dtype),
    )(xr)
    return out.reshape(x.shape)


# speedup vs baseline: 2.5285x; 2.5285x over previous
import jax
import jax.numpy as jnp
from jax import lax
from jax.experimental import pallas as pl

_UNK = 22
_THRESH = 838861
_K0 = 0
_K1 = 42
_KS2 = _K0 ^ _K1 ^ 0x1BD11BDA
_ROT = ((13, 15, 26, 6), (17, 29, 16, 24))


def _rotl(v, d):
    return lax.shift_right_logical(v, jnp.int32(32 - d)) | (v << jnp.int32(d))


def _threefry_bits(idx):
    x0 = jnp.full(idx.shape, jnp.int32(_K0), dtype=jnp.int32)
    x1 = idx + jnp.int32(_K1)
    ks = (_K0, _K1, _KS2)
    for i in range(5):
        for d in _ROT[i % 2]:
            x0 = x0 + x1
            x1 = _rotl(x1, d)
            x1 = x1 ^ x0
        x0 = x0 + jnp.int32(ks[(i + 1) % 3])
        x1 = x1 + jnp.int32((ks[(i + 2) % 3] + i + 1) & 0x7FFFFFFF) + jnp.int32(
            -0x80000000 if (ks[(i + 2) % 3] + i + 1) & 0x80000000 else 0
        )
    return x0 ^ x1


def _threefry_mask_body(x_ref, o_ref):
    x = x_ref[...]
    rows, cols = x.shape
    half = cols // 2
    ps = (2 * rows, half)
    s = lax.broadcasted_iota(jnp.int32, ps, 0)
    l = lax.broadcasted_iota(jnp.int32, ps, 1)
    idx = (s & jnp.int32(rows - 1)) * jnp.int32(cols) + (
        lax.shift_right_logical(s, jnp.int32(rows.bit_length() - 1)) * jnp.int32(half)
    ) + l
    bits = _threefry_bits(idx)
    m8 = lax.shift_right_logical(bits, jnp.int32(9)) < jnp.int32(_THRESH)
    mask = jnp.concatenate([m8[:rows, :], m8[rows:, :]], axis=1)
    o_ref[...] = jnp.where(mask, jnp.int32(_UNK), x)


@jax.jit
def kernel(x):
    return pl.pallas_call(
        _threefry_mask_body,
        out_shape=jax.ShapeDtypeStruct(x.shape, x.dtype),
    )(x)
